# baseline (speedup 1.0000x reference)
import functools

import jax
import jax.numpy as jnp
from jax import lax
from jax.experimental import pallas as pl
from jax.experimental.pallas import tpu as pltpu
from jax.experimental.pallas import tpu_sc as plsc

_K = 64
_D = 32
_N = _K * _K * _K * _D
_NC = 2
_NS = 16
_NW = _NC * _NS
_PER_W = _N // _NW
_BUF = 16384
_NDMA = _PER_W // _BUF


def _fill_body(lat_hbm, c0_hbm, c1_hbm, c2_hbm, out_hbm, vecs_v, buf_v,
               shared_v, sem):
    cid = lax.axis_index("c")
    sid = lax.axis_index("s")
    wid = cid * _NS + sid

    pltpu.sync_copy(lat_hbm, vecs_v.at[0])
    pltpu.sync_copy(c0_hbm, vecs_v.at[1])
    pltpu.sync_copy(c1_hbm, vecs_v.at[2])
    pltpu.sync_copy(c2_hbm, vecs_v.at[3])

    chunks = []
    for j in range(_D // 16):
        s = pl.ds(16 * j, 16)
        chunks.append(3.0 * vecs_v[0, s] - 2.0 * vecs_v[1, s]
                      - vecs_v[2, s] + vecs_v[3, s])

    def fill(i, carry):
        base = i * 64
        buf_v[pl.ds(base, 16)] = chunks[0]
        buf_v[pl.ds(base + 16, 16)] = chunks[1]
        buf_v[pl.ds(base + 32, 16)] = chunks[0]
        buf_v[pl.ds(base + 48, 16)] = chunks[1]
        return carry

    lax.fori_loop(0, _BUF // 64, fill, 0)

    @pl.when(sid == 0)
    def _():
        pltpu.sync_copy(buf_v, shared_v)

    plsc.subcore_barrier()

    base = wid * _PER_W
    copies = [
        pltpu.async_copy(shared_v, out_hbm.at[pl.ds(base + i * _BUF, _BUF)],
                         sem)
        for i in range(_NDMA)
    ]
    for c in copies:
        c.wait()


@functools.partial(jax.jit, static_argnums=())
def _broadcast_fill(lat, c0, c1, c2):
    mesh = plsc.VectorSubcoreMesh(core_axis_name="c", subcore_axis_name="s")
    f = functools.partial(
        pl.kernel,
        mesh=mesh,
        out_type=jax.ShapeDtypeStruct((_N,), jnp.float32),
        scratch_types=[
            pltpu.VMEM((4, _D), jnp.float32),
            pltpu.VMEM((_BUF,), jnp.float32),
            pltpu.VMEM_SHARED((_BUF,), jnp.float32),
            pltpu.SemaphoreType.DMA,
        ],
    )(_fill_body)
    return f(lat, c0, c1, c2)


def kernel(latent_representation, codebook0, codebook1, codebook2):
    out = _broadcast_fill(
        latent_representation.reshape(_D),
        codebook0[0],
        codebook1[0],
        codebook2[0],
    )
    return out.reshape(1, _K, _K, _K, _D)


# speedup vs baseline: 6.2401x; 6.2401x over previous
import functools

import jax
import jax.numpy as jnp
from jax.experimental import pallas as pl

_K = 64
_D = 32
_BLK = 4


def _fill_body(lat_ref, c0_ref, c1_ref, c2_ref, out_ref):
    v = (3.0 * lat_ref[0:1, :] - 2.0 * c0_ref[0:1, :]
         - c1_ref[0:1, :] + c2_ref[0:1, :])
    out_ref[...] = jnp.broadcast_to(
        v.reshape(1, 1, 1, _D, 1), out_ref.shape)


def _broadcast_fill(lat, c0, c1, c2):
    return pl.pallas_call(
        _fill_body,
        grid=(_K // _BLK,),
        in_specs=[
            pl.BlockSpec((1, _D), lambda i: (0, 0)),
            pl.BlockSpec((_K, _D), lambda i: (0, 0)),
            pl.BlockSpec((_K, _D), lambda i: (0, 0)),
            pl.BlockSpec((_K, _D), lambda i: (0, 0)),
        ],
        out_specs=pl.BlockSpec((1, _BLK, _K, _D, _K),
                               lambda i: (0, i, 0, 0, 0)),
        out_shape=jax.ShapeDtypeStruct((1, _K, _K, _D, _K), jnp.float32),
    )(lat, c0, c1, c2)


def kernel(latent_representation, codebook0, codebook1, codebook2):
    out = _broadcast_fill(
        latent_representation, codebook0, codebook1, codebook2)
    return jnp.transpose(out, (0, 1, 2, 4, 3))
